# baseline (speedup 1.0000x reference)
import functools

import jax
import jax.numpy as jnp
from jax import lax
from jax.experimental import pallas as pl
from jax.experimental.pallas import tpu as pltpu
from jax.experimental.pallas import tpu_sc as plsc

_NC, _NS, _LANES = 2, 16, 16
_NW = _NC * _NS
_CH = 512


def _sc_col_sums(xT, emb_table):
    L, B = xT.shape
    _, DIM = emb_table.shape
    nch = B // _CH
    n_iter = (L + _NW - 1) // _NW

    mesh = plsc.VectorSubcoreMesh(core_axis_name="c", subcore_axis_name="s")

    @functools.partial(
        pl.kernel,
        out_type=jax.ShapeDtypeStruct((L, DIM), jnp.float32),
        mesh=mesh,
        scratch_types=[
            pltpu.VMEM((B,), jnp.int32),
            pltpu.VMEM((_CH, DIM), jnp.float32),
            pltpu.VMEM((_CH, DIM), jnp.float32),
            pltpu.VMEM((DIM,), jnp.float32),
            pltpu.SemaphoreType.DMA,
            pltpu.SemaphoreType.DMA,
        ],
        compiler_params=pltpu.CompilerParams(use_tc_tiling_on_sc=False),
    )
    def k(xT_hbm, table_hbm, out_hbm, idx_v, acc_a, acc_b, row_v, sem_a, sem_b):
        wid = lax.axis_index("s") * _NC + lax.axis_index("c")

        def body_i(i, carry):
            l = i * _NW + wid

            @pl.when(l < L)
            def _():
                pltpu.sync_copy(xT_hbm.at[l], idx_v)
                pltpu.async_copy(
                    table_hbm.at[idx_v.at[pl.ds(0, _CH)]], acc_a, sem_a)
                pltpu.async_copy(
                    table_hbm.at[idx_v.at[pl.ds(_CH, _CH)]], acc_b, sem_b)

                def pair(p, c2):
                    pltpu.make_async_copy(
                        table_hbm.at[idx_v.at[pl.ds(0, _CH)]], acc_a, sem_a
                    ).wait()
                    pltpu.async_copy(
                        table_hbm.at[idx_v.at[pl.ds(c2 * _CH, _CH)]],
                        acc_a, sem_a, add=True)
                    pltpu.make_async_copy(
                        table_hbm.at[idx_v.at[pl.ds(0, _CH)]], acc_b, sem_b
                    ).wait()
                    pltpu.async_copy(
                        table_hbm.at[idx_v.at[pl.ds((c2 + 1) * _CH, _CH)]],
                        acc_b, sem_b, add=True)
                    return c2 + 2

                lax.fori_loop(1, nch // 2, pair, 2)
                pltpu.make_async_copy(
                    table_hbm.at[idx_v.at[pl.ds(0, _CH)]], acc_a, sem_a).wait()
                pltpu.make_async_copy(
                    table_hbm.at[idx_v.at[pl.ds(0, _CH)]], acc_b, sem_b).wait()

                def red(g, acc):
                    a0, a1 = acc
                    a0 = a0 + acc_a[g, pl.ds(0, _LANES)] + acc_b[g, pl.ds(0, _LANES)]
                    a1 = a1 + acc_a[g, pl.ds(_LANES, _LANES)] + acc_b[g, pl.ds(_LANES, _LANES)]
                    return (a0, a1)

                z = jnp.zeros((_LANES,), jnp.float32)
                a0, a1 = lax.fori_loop(0, _CH, red, (z, z), unroll=8)
                row_v[pl.ds(0, _LANES)] = a0
                row_v[pl.ds(_LANES, _LANES)] = a1
                pltpu.sync_copy(row_v, out_hbm.at[l])

            return carry

        lax.fori_loop(0, n_iter, body_i, 0)

    return k(xT, emb_table)


def _tc_transpose(x):
    B, L = x.shape
    blk = 2048

    def body(x_ref, o_ref):
        o_ref[...] = x_ref[...].T

    return pl.pallas_call(
        body,
        grid=(B // blk,),
        in_specs=[pl.BlockSpec((blk, L), lambda i: (i, 0))],
        out_specs=pl.BlockSpec((L, blk), lambda i: (0, i)),
        out_shape=jax.ShapeDtypeStruct((L, B), jnp.int32),
    )(x)


def _tc_finish(sums, W, b2d, n_total):
    def body(sums_ref, w_ref, b_ref, out_ref):
        m = sums_ref[...] * (1.0 / n_total)
        out_ref[...] = lax.dot_general(
            m, w_ref[...], (((1,), (1,)), ((), ())),
            preferred_element_type=jnp.float32) + b_ref[...]

    L, DIM = sums.shape
    return pl.pallas_call(
        body,
        out_shape=jax.ShapeDtypeStruct((L, DIM), jnp.float32),
    )(sums, W, b2d)


def kernel(x, emb_table, W, b):
    x = x.astype(jnp.int32)
    B, L = x.shape
    xT = _tc_transpose(x)
    sums = _sc_col_sums(xT, emb_table)
    return _tc_finish(sums, W, b.reshape(1, -1), B)


# speedup vs baseline: 1.0194x; 1.0194x over previous
import functools

import jax
import jax.numpy as jnp
from jax import lax
from jax.experimental import pallas as pl
from jax.experimental.pallas import tpu as pltpu
from jax.experimental.pallas import tpu_sc as plsc

_NC, _NS, _LANES = 2, 16, 16
_NW = _NC * _NS
_CH = 512
_NBUF = 4
_RCH = 128


def _sc_transpose(x):
    B, L = x.shape
    bpw = B // _NW
    nrch = bpw // _RCH
    nl16 = (L + _LANES - 1) // _LANES
    lpad = nl16 * _LANES

    mesh = plsc.VectorSubcoreMesh(core_axis_name="c", subcore_axis_name="s")

    @functools.partial(
        pl.kernel,
        out_type=jax.ShapeDtypeStruct((L, B), jnp.int32),
        mesh=mesh,
        scratch_types=[
            pltpu.VMEM((_RCH, lpad), jnp.int32),
            pltpu.VMEM((L, bpw), jnp.int32),
        ],
        compiler_params=pltpu.CompilerParams(
            use_tc_tiling_on_sc=False, needs_layout_passes=False),
    )
    def k(x_hbm, xT_hbm, in_v, col_v):
        wid = lax.axis_index("s") * _NC + lax.axis_index("c")
        base = wid * bpw
        lane = lax.iota(jnp.int32, _LANES)

        def chunk(c, carry):
            pltpu.sync_copy(x_hbm.at[pl.ds(base + c * _RCH, _RCH)],
                            in_v.at[:, pl.ds(0, L)])

            def row(b, _):
                col = jnp.full((_LANES,), c * _RCH + b, jnp.int32)
                for li in range(nl16):
                    vals = in_v[b, pl.ds(li * _LANES, _LANES)]
                    lrow = lane + (li * _LANES)
                    if (li + 1) * _LANES <= L:
                        plsc.store_scatter(col_v, [lrow, col], vals)
                    else:
                        plsc.store_scatter(col_v, [lrow, col], vals,
                                           mask=lrow < L)
                return _

            lax.fori_loop(0, _RCH, row, 0)
            return carry

        lax.fori_loop(0, nrch, chunk, 0)
        pltpu.sync_copy(col_v, xT_hbm.at[:, pl.ds(base, bpw)])

    return k(x)


def _sc_col_sums(xT, emb_table):
    L, B = xT.shape
    _, DIM = emb_table.shape
    nch = B // _CH
    n_iter = (L + _NW - 1) // _NW

    mesh = plsc.VectorSubcoreMesh(core_axis_name="c", subcore_axis_name="s")

    @functools.partial(
        pl.kernel,
        out_type=jax.ShapeDtypeStruct((L, DIM), jnp.float32),
        mesh=mesh,
        scratch_types=[
            pltpu.VMEM((B,), jnp.int32),
            [pltpu.VMEM((_CH, DIM), jnp.float32) for _ in range(_NBUF)],
            pltpu.VMEM((DIM,), jnp.float32),
            [pltpu.SemaphoreType.DMA for _ in range(_NBUF)],
        ],
        compiler_params=pltpu.CompilerParams(use_tc_tiling_on_sc=False),
    )
    def k(xT_hbm, table_hbm, out_hbm, idx_v, accs, row_v, sems):
        wid = lax.axis_index("s") * _NC + lax.axis_index("c")

        def body_i(i, carry):
            l = i * _NW + wid

            @pl.when(l < L)
            def _():
                pltpu.sync_copy(xT_hbm.at[l], idx_v)
                for n in range(_NBUF):
                    pltpu.async_copy(
                        table_hbm.at[idx_v.at[pl.ds(n * _CH, _CH)]],
                        accs[n], sems[n])

                def grp(p, c0):
                    for n in range(_NBUF):
                        pltpu.make_async_copy(
                            table_hbm.at[idx_v.at[pl.ds(0, _CH)]],
                            accs[n], sems[n]).wait()
                        pltpu.async_copy(
                            table_hbm.at[idx_v.at[pl.ds((c0 + n) * _CH, _CH)]],
                            accs[n], sems[n], add=True)
                    return c0 + _NBUF

                lax.fori_loop(1, nch // _NBUF, grp, _NBUF)
                for n in range(_NBUF):
                    pltpu.make_async_copy(
                        table_hbm.at[idx_v.at[pl.ds(0, _CH)]],
                        accs[n], sems[n]).wait()

                def red(g, acc):
                    a0, a1 = acc
                    for n in range(_NBUF):
                        a0 = a0 + accs[n][g, pl.ds(0, _LANES)]
                        a1 = a1 + accs[n][g, pl.ds(_LANES, _LANES)]
                    return (a0, a1)

                z = jnp.zeros((_LANES,), jnp.float32)
                a0, a1 = lax.fori_loop(0, _CH, red, (z, z), unroll=4)
                row_v[pl.ds(0, _LANES)] = a0
                row_v[pl.ds(_LANES, _LANES)] = a1
                pltpu.sync_copy(row_v, out_hbm.at[l])

            return carry

        lax.fori_loop(0, n_iter, body_i, 0)

    return k(xT, emb_table)


def _tc_finish(sums, W, b2d, n_total):
    def body(sums_ref, w_ref, b_ref, out_ref):
        m = sums_ref[...] * (1.0 / n_total)
        out_ref[...] = lax.dot_general(
            m, w_ref[...], (((1,), (1,)), ((), ())),
            preferred_element_type=jnp.float32) + b_ref[...]

    L, DIM = sums.shape
    return pl.pallas_call(
        body,
        out_shape=jax.ShapeDtypeStruct((L, DIM), jnp.float32),
    )(sums, W, b2d)


def kernel(x, emb_table, W, b):
    x = x.astype(jnp.int32)
    B, L = x.shape
    xT = _sc_transpose(x)
    sums = _sc_col_sums(xT, emb_table)
    return _tc_finish(sums, W, b.reshape(1, -1), B)


# speedup vs baseline: 1.0202x; 1.0008x over previous
import functools

import jax
import jax.numpy as jnp
from jax import lax
from jax.experimental import pallas as pl
from jax.experimental.pallas import tpu as pltpu
from jax.experimental.pallas import tpu_sc as plsc

_NC, _NS, _LANES = 2, 16, 16
_NW = _NC * _NS
_CH = 512
_NBUF = 4
_RCH = 128


def _sc_transpose(x_flat, B, L):
    bpw = B // _NW
    nrch = bpw // _RCH
    nl16 = (L + _LANES - 1) // _LANES

    mesh = plsc.VectorSubcoreMesh(core_axis_name="c", subcore_axis_name="s")

    @functools.partial(
        pl.kernel,
        out_type=jax.ShapeDtypeStruct((L, B), jnp.int32),
        mesh=mesh,
        scratch_types=[
            pltpu.VMEM((_RCH * L + _LANES,), jnp.int32),
            pltpu.VMEM((L, bpw), jnp.int32),
        ],
        compiler_params=pltpu.CompilerParams(
            use_tc_tiling_on_sc=False, needs_layout_passes=False),
    )
    def k(x_hbm, xT_hbm, in_v, col_v):
        wid = lax.axis_index("s") * _NC + lax.axis_index("c")
        base = wid * bpw
        lane = lax.iota(jnp.int32, _LANES)

        def chunk(c, carry):
            pltpu.sync_copy(x_hbm.at[pl.ds((base + c * _RCH) * L, _RCH * L)],
                            in_v.at[pl.ds(0, _RCH * L)])

            def row(b, _):
                col = jnp.full((_LANES,), c * _RCH + b, jnp.int32)
                for li in range(nl16):
                    vals = in_v[pl.ds(b * L + li * _LANES, _LANES)]
                    lrow = lane + (li * _LANES)
                    if (li + 1) * _LANES <= L:
                        plsc.store_scatter(col_v, [lrow, col], vals)
                    else:
                        plsc.store_scatter(col_v, [lrow, col], vals,
                                           mask=lrow < L)
                return _

            lax.fori_loop(0, _RCH, row, 0)
            return carry

        lax.fori_loop(0, nrch, chunk, 0)
        pltpu.sync_copy(col_v, xT_hbm.at[:, pl.ds(base, bpw)])

    return k(x_flat)


def _sc_col_sums(xT, emb_table):
    L, B = xT.shape
    _, DIM = emb_table.shape
    nch = B // _CH
    n_iter = (L + _NW - 1) // _NW

    mesh = plsc.VectorSubcoreMesh(core_axis_name="c", subcore_axis_name="s")

    @functools.partial(
        pl.kernel,
        out_type=jax.ShapeDtypeStruct((L, DIM), jnp.float32),
        mesh=mesh,
        scratch_types=[
            pltpu.VMEM((B,), jnp.int32),
            [pltpu.VMEM((_CH, DIM), jnp.float32) for _ in range(_NBUF)],
            pltpu.VMEM((DIM,), jnp.float32),
            [pltpu.SemaphoreType.DMA for _ in range(_NBUF)],
        ],
        compiler_params=pltpu.CompilerParams(use_tc_tiling_on_sc=False),
    )
    def k(xT_hbm, table_hbm, out_hbm, idx_v, accs, row_v, sems):
        wid = lax.axis_index("s") * _NC + lax.axis_index("c")

        def body_i(i, carry):
            l = i * _NW + wid

            @pl.when(l < L)
            def _():
                pltpu.sync_copy(xT_hbm.at[l], idx_v)
                for n in range(_NBUF):
                    pltpu.async_copy(
                        table_hbm.at[idx_v.at[pl.ds(n * _CH, _CH)]],
                        accs[n], sems[n])

                def grp(p, c0):
                    for n in range(_NBUF):
                        pltpu.make_async_copy(
                            table_hbm.at[idx_v.at[pl.ds(0, _CH)]],
                            accs[n], sems[n]).wait()
                        pltpu.async_copy(
                            table_hbm.at[idx_v.at[pl.ds((c0 + n) * _CH, _CH)]],
                            accs[n], sems[n], add=True)
                    return c0 + _NBUF

                lax.fori_loop(1, nch // _NBUF, grp, _NBUF)
                for n in range(_NBUF):
                    pltpu.make_async_copy(
                        table_hbm.at[idx_v.at[pl.ds(0, _CH)]],
                        accs[n], sems[n]).wait()

                def red(g, acc):
                    a0, a1 = acc
                    for n in range(_NBUF):
                        a0 = a0 + accs[n][g, pl.ds(0, _LANES)]
                        a1 = a1 + accs[n][g, pl.ds(_LANES, _LANES)]
                    return (a0, a1)

                z = jnp.zeros((_LANES,), jnp.float32)
                a0, a1 = lax.fori_loop(0, _CH, red, (z, z), unroll=4)
                row_v[pl.ds(0, _LANES)] = a0
                row_v[pl.ds(_LANES, _LANES)] = a1
                pltpu.sync_copy(row_v, out_hbm.at[l])

            return carry

        lax.fori_loop(0, n_iter, body_i, 0)

    return k(xT, emb_table)


def _tc_finish(sums, W, b2d, n_total):
    def body(sums_ref, w_ref, b_ref, out_ref):
        m = sums_ref[...] * (1.0 / n_total)
        out_ref[...] = lax.dot_general(
            m, w_ref[...], (((1,), (1,)), ((), ())),
            preferred_element_type=jnp.float32) + b_ref[...]

    L, DIM = sums.shape
    return pl.pallas_call(
        body,
        out_shape=jax.ShapeDtypeStruct((L, DIM), jnp.float32),
    )(sums, W, b2d)


def kernel(x, emb_table, W, b):
    B, L = x.shape
    x_flat = jnp.reshape(x.astype(jnp.int32), (B * L,))
    xT = _sc_transpose(x_flat, B, L)
    sums = _sc_col_sums(xT, emb_table)
    return _tc_finish(sums, W, b.reshape(1, -1), B)
